# baseline (speedup 1.0000x reference)
import functools
import math

import jax
import jax.numpy as jnp
from jax import lax
from jax.experimental import pallas as pl
from jax.experimental.pallas import tpu as pltpu
from jax.experimental.pallas import tpu_sc as plsc

_NUM_HEADS = 4
_DIMS = 32
_D = _DIMS * _NUM_HEADS
_B, _N, _P = 8, 4096, 64

_NW = 32
_CHUNK = 512
_LANES = 16
_NJ = _D // _LANES

_NHALF = 1
_BH = _B // _NHALF
_ROWS_H = _BH * _N
_RPW = _ROWS_H // _NW
_NCHUNK = _RPW // _CHUNK
_WPB = _NW // _BH


def _lane_shift(v, d):
    if d == 0:
        return v
    z = jnp.zeros((2, abs(d)), v.dtype)
    if d > 0:
        return jnp.concatenate([v[:, d:], z], axis=1)
    return jnp.concatenate([z, v[:, :d]], axis=1)


def _xw_body(c0_ref, c1_ref, x_ref, wc_ref, b1_ref, l2w_ref, l2b_ref,
             rep_ref, o_ref):
    ct = jnp.concatenate([c0_ref[0], c1_ref[0]], axis=0)
    x = x_ref[0]
    s = jnp.concatenate([_lane_shift(ct, o) for o in range(-2, 3)], axis=0)
    h = lax.dot_general(s, wc_ref[...], (((0,), (0,)), ((), ())),
                        preferred_element_type=jnp.float32)
    h = h + b1_ref[...]
    h = 0.5 * h * (1.0 + lax.erf(h * (1.0 / math.sqrt(2.0))))
    w = jnp.dot(h, l2w_ref[...], preferred_element_type=jnp.float32)
    w = w + l2b_ref[...]
    xr = jnp.dot(x, rep_ref[...], preferred_element_type=jnp.float32)
    o_ref[0] = (xr * w).astype(jnp.bfloat16)


_xw_call = pl.pallas_call(
    _xw_body,
    grid=(_BH,),
    in_specs=[
        pl.BlockSpec((1, 1, _N), lambda b: (b, 0, 0)),
        pl.BlockSpec((1, 1, _N), lambda b: (b, 0, 0)),
        pl.BlockSpec((1, _N, _DIMS), lambda b: (b, 0, 0)),
        pl.BlockSpec((10, _D), lambda b: (0, 0)),
        pl.BlockSpec((1, _D), lambda b: (0, 0)),
        pl.BlockSpec((_D, _D), lambda b: (0, 0)),
        pl.BlockSpec((1, _D), lambda b: (0, 0)),
        pl.BlockSpec((_DIMS, _D), lambda b: (0, 0)),
    ],
    out_specs=pl.BlockSpec((1, _N, _D), lambda b: (b, 0, 0)),
    out_shape=jax.ShapeDtypeStruct((_BH, _N, _D), jnp.bfloat16),
)


@functools.partial(
    pl.kernel,
    out_type=(
        jax.ShapeDtypeStruct((_NW, _P * _D), jnp.float32),
        jax.ShapeDtypeStruct((_NW, _P * _LANES), jnp.float32),
    ),
    mesh=plsc.VectorSubcoreMesh(core_axis_name="c", subcore_axis_name="s"),
    scratch_types=[
        pltpu.VMEM((_RPW,), jnp.int32),
        pltpu.VMEM((_CHUNK // 2, 2, _D), jnp.bfloat16),
        pltpu.VMEM((_CHUNK // 2, 2, _D), jnp.bfloat16),
        pltpu.VMEM((_P * _D,), jnp.float32),
        pltpu.VMEM((_P * _LANES,), jnp.float32),
        pltpu.SemaphoreType.DMA,
        pltpu.SemaphoreType.DMA,
    ],
)
def _seg_partial(xw_hbm, seg_hbm, ps_hbm, cnt_hbm, segv, buf0, buf1, acc,
                 cntl, sem0, sem1):
    wid = lax.axis_index("c") * 16 + lax.axis_index("s")
    base = wid * _RPW
    base2 = wid * (_RPW // 2)
    zeros16 = jnp.zeros((_LANES,), jnp.float32)
    ones16 = jnp.ones((_LANES,), jnp.float32)
    bufs = (buf0, buf1)
    sems = (sem0, sem1)

    zeros128 = jnp.zeros((_D,), jnp.float32)

    def zrow(i, carry):
        acc[pl.ds(i * _D, _D)] = zeros128
        cntl[pl.ds(i * _LANES, _LANES)] = zeros16
        return carry

    lax.fori_loop(0, _P, zrow, 0)

    pltpu.sync_copy(seg_hbm.at[pl.ds(base, _RPW)], segv)
    pending = pltpu.async_copy(xw_hbm.at[pl.ds(base2, _CHUNK // 2)], buf0,
                               sem0)

    sixteen16 = jnp.full((_LANES,), 16.0, jnp.float32)

    for g in range(_NCHUNK):
        if g + 1 < _NCHUNK:
            nxt = pltpu.async_copy(
                xw_hbm.at[pl.ds(base2 + (g + 1) * (_CHUNK // 2), _CHUNK // 2)],
                bufs[(g + 1) % 2], sems[(g + 1) % 2])
        pending.wait()
        buf = bufs[g % 2]

        def group(t, carry):
            sv = segv[pl.ds(g * _CHUNK + t * _LANES, _LANES)]
            s0 = sv[0]
            s15 = sv[_LANES - 1]

            rows = []
            for i in range(_LANES):
                wide = buf[t * (_LANES // 2) + i // 2, i % 2,
                           :].astype(jnp.float32)
                rows.append([lax.slice(wide, (j * _LANES,),
                                       ((j + 1) * _LANES,))
                             for j in range(_NJ)])

            @pl.when(s0 == s15)
            def _uniform():
                for j in range(_NJ):
                    vals = [rows[i][j] for i in range(_LANES)]
                    while len(vals) > 1:
                        vals = [vals[k] + vals[k + 1]
                                for k in range(0, len(vals), 2)]
                    plsc.addupdate(
                        acc.at[pl.ds(s0 * _D + j * _LANES, _LANES)], vals[0])
                plsc.addupdate(cntl.at[pl.ds(s0 * _LANES, _LANES)], sixteen16)

            @pl.when(s0 != s15)
            def _boundary():
                for i in range(_LANES):
                    si = sv[i]
                    for j in range(_NJ):
                        plsc.addupdate(
                            acc.at[pl.ds(si * _D + j * _LANES, _LANES)],
                            rows[i][j])
                    plsc.addupdate(cntl.at[pl.ds(si * _LANES, _LANES)],
                                   ones16)
            return carry

        lax.fori_loop(0, _CHUNK // _LANES, group, 0)
        if g + 1 < _NCHUNK:
            pending = nxt

    pltpu.sync_copy(acc, ps_hbm.at[wid])
    pltpu.sync_copy(cntl, cnt_hbm.at[wid])


def _wave_body(w1_ref, w2_ref, ps_ref, cnt_ref, o_ref):
    sums = jnp.sum(ps_ref[...], axis=0)
    c = jnp.sum(cnt_ref[...], axis=0)[:, 0:1]
    mean = sums / c
    o_ref[0] = w1_ref[0] * jnp.sin(mean) + w2_ref[0] * jnp.cos(mean)


_wave_call = pl.pallas_call(
    _wave_body,
    grid=(_BH,),
    in_specs=[
        pl.BlockSpec(memory_space=pltpu.SMEM),
        pl.BlockSpec(memory_space=pltpu.SMEM),
        pl.BlockSpec((_WPB, _P, _D), lambda b: (b, 0, 0)),
        pl.BlockSpec((_WPB, _P, _LANES), lambda b: (b, 0, 0)),
    ],
    out_specs=pl.BlockSpec((1, _P, _D), lambda b: (b, 0, 0)),
    out_shape=jax.ShapeDtypeStruct((_BH, _P, _D), jnp.float32),
)


def kernel(x, coords, indices, patch_seq_len, conv_w, conv_b, lin1_w, lin1_b,
           lin2_w, lin2_b, wave_w1, wave_w2):
    cw = conv_w[:, 0, :]
    taps = cw + jnp.zeros((2, 5), cw.dtype).at[:, 2].set(1.0)
    wc = taps.T.reshape(10, 1) * jnp.tile(lin1_w, (5, 1))
    b1 = (conv_b @ lin1_w + lin1_b).reshape(1, _D)
    l2w = lin2_w
    l2b = lin2_b.reshape(1, _D)
    rep = jnp.tile(jnp.eye(_DIMS, dtype=jnp.float32), (1, _NUM_HEADS))

    seg = indices[..., 0].astype(jnp.int32)
    seg = seg - (jnp.asarray(patch_seq_len, jnp.int32) - _P)

    c0 = coords[:, :, 0].reshape(_B, 1, _N)
    c1 = coords[:, :, 1].reshape(_B, 1, _N)
    xw = _xw_call(c0, c1, x, wc, b1, l2w, l2b, rep)
    ps, cnt = _seg_partial(xw.reshape(_ROWS_H // 2, 2, _D),
                           seg.reshape(_ROWS_H))
    ps = ps.reshape(_NW, _P, _D)
    cnt = cnt.reshape(_NW, _P, _LANES)
    return _wave_call(wave_w1, wave_w2, ps, cnt)


# speedup vs baseline: 1.0192x; 1.0192x over previous
import functools
import math

import jax
import jax.numpy as jnp
from jax import lax
from jax.experimental import pallas as pl
from jax.experimental.pallas import tpu as pltpu
from jax.experimental.pallas import tpu_sc as plsc

_NUM_HEADS = 4
_DIMS = 32
_D = _DIMS * _NUM_HEADS
_B, _N, _P = 8, 4096, 64

_NW = 32
_CHUNK = 256
_LANES = 16
_NJ = _D // _LANES

_NHALF = 1
_BH = _B // _NHALF
_ROWS_H = _BH * _N
_RPW = _ROWS_H // _NW
_NCHUNK = _RPW // _CHUNK
_WPB = _NW // _BH


def _lane_shift(v, d):
    if d == 0:
        return v
    z = jnp.zeros((2, abs(d)), v.dtype)
    if d > 0:
        return jnp.concatenate([v[:, d:], z], axis=1)
    return jnp.concatenate([z, v[:, :d]], axis=1)


def _xw_body(c0_ref, c1_ref, x_ref, wc_ref, b1_ref, l2w_ref, l2b_ref,
             rep_ref, o_ref):
    ct = jnp.concatenate([c0_ref[0], c1_ref[0]], axis=0)
    x = x_ref[0]
    s = jnp.concatenate([_lane_shift(ct, o) for o in range(-2, 3)], axis=0)
    h = lax.dot_general(s, wc_ref[...], (((0,), (0,)), ((), ())),
                        preferred_element_type=jnp.float32)
    h = h + b1_ref[...]
    h = 0.5 * h * (1.0 + lax.erf(h * (1.0 / math.sqrt(2.0))))
    w = jnp.dot(h, l2w_ref[...], preferred_element_type=jnp.float32)
    w = w + l2b_ref[...]
    xr = jnp.dot(x, rep_ref[...], preferred_element_type=jnp.float32)
    o_ref[0] = (xr * w).astype(jnp.bfloat16)


_xw_call = pl.pallas_call(
    _xw_body,
    grid=(_BH,),
    in_specs=[
        pl.BlockSpec((1, 1, _N), lambda b: (b, 0, 0)),
        pl.BlockSpec((1, 1, _N), lambda b: (b, 0, 0)),
        pl.BlockSpec((1, _N, _DIMS), lambda b: (b, 0, 0)),
        pl.BlockSpec((10, _D), lambda b: (0, 0)),
        pl.BlockSpec((1, _D), lambda b: (0, 0)),
        pl.BlockSpec((_D, _D), lambda b: (0, 0)),
        pl.BlockSpec((1, _D), lambda b: (0, 0)),
        pl.BlockSpec((_DIMS, _D), lambda b: (0, 0)),
    ],
    out_specs=pl.BlockSpec((1, _N, _D), lambda b: (b, 0, 0)),
    out_shape=jax.ShapeDtypeStruct((_BH, _N, _D), jnp.bfloat16),
)


@functools.partial(
    pl.kernel,
    out_type=(
        jax.ShapeDtypeStruct((_NW, _P * _D), jnp.float32),
        jax.ShapeDtypeStruct((_NW, _P * _LANES), jnp.float32),
    ),
    mesh=plsc.VectorSubcoreMesh(core_axis_name="c", subcore_axis_name="s"),
    scratch_types=[
        pltpu.VMEM((_RPW,), jnp.int32),
        pltpu.VMEM((_CHUNK // 2, 2, _D), jnp.bfloat16),
        pltpu.VMEM((_CHUNK // 2, 2, _D), jnp.bfloat16),
        pltpu.VMEM((_P * _D,), jnp.float32),
        pltpu.VMEM((_P * _LANES,), jnp.float32),
        pltpu.SemaphoreType.DMA,
        pltpu.SemaphoreType.DMA,
    ],
)
def _seg_partial(xw_hbm, seg_hbm, ps_hbm, cnt_hbm, segv, buf0, buf1, acc,
                 cntl, sem0, sem1):
    wid = lax.axis_index("c") * 16 + lax.axis_index("s")
    base = wid * _RPW
    base2 = wid * (_RPW // 2)
    zeros16 = jnp.zeros((_LANES,), jnp.float32)
    ones16 = jnp.ones((_LANES,), jnp.float32)
    bufs = (buf0, buf1)
    sems = (sem0, sem1)

    zeros128 = jnp.zeros((_D,), jnp.float32)

    def zrow(i, carry):
        acc[pl.ds(i * _D, _D)] = zeros128
        cntl[pl.ds(i * _LANES, _LANES)] = zeros16
        return carry

    lax.fori_loop(0, _P, zrow, 0)

    pltpu.sync_copy(seg_hbm.at[pl.ds(base, _RPW)], segv)
    pending = pltpu.async_copy(xw_hbm.at[pl.ds(base2, _CHUNK // 2)], buf0,
                               sem0)

    sixteen16 = jnp.full((_LANES,), 16.0, jnp.float32)

    for g in range(_NCHUNK):
        if g + 1 < _NCHUNK:
            nxt = pltpu.async_copy(
                xw_hbm.at[pl.ds(base2 + (g + 1) * (_CHUNK // 2), _CHUNK // 2)],
                bufs[(g + 1) % 2], sems[(g + 1) % 2])
        pending.wait()
        buf = bufs[g % 2]

        def group(t, carry):
            sv = segv[pl.ds(g * _CHUNK + t * _LANES, _LANES)]
            s0 = sv[0]
            s15 = sv[_LANES - 1]

            rows = []
            for i in range(_LANES):
                wide = buf[t * (_LANES // 2) + i // 2, i % 2,
                           :].astype(jnp.float32)
                rows.append([lax.slice(wide, (j * _LANES,),
                                       ((j + 1) * _LANES,))
                             for j in range(_NJ)])

            @pl.when(s0 == s15)
            def _uniform():
                for j in range(_NJ):
                    vals = [rows[i][j] for i in range(_LANES)]
                    while len(vals) > 1:
                        vals = [vals[k] + vals[k + 1]
                                for k in range(0, len(vals), 2)]
                    plsc.addupdate(
                        acc.at[pl.ds(s0 * _D + j * _LANES, _LANES)], vals[0])
                plsc.addupdate(cntl.at[pl.ds(s0 * _LANES, _LANES)], sixteen16)

            @pl.when(s0 != s15)
            def _boundary():
                for i in range(_LANES):
                    si = sv[i]
                    for j in range(_NJ):
                        plsc.addupdate(
                            acc.at[pl.ds(si * _D + j * _LANES, _LANES)],
                            rows[i][j])
                    plsc.addupdate(cntl.at[pl.ds(si * _LANES, _LANES)],
                                   ones16)
            return carry

        lax.fori_loop(0, _CHUNK // _LANES, group, 0)
        if g + 1 < _NCHUNK:
            pending = nxt

    pltpu.sync_copy(acc, ps_hbm.at[wid])
    pltpu.sync_copy(cntl, cnt_hbm.at[wid])


def _wave_body(w1_ref, w2_ref, ps_ref, cnt_ref, o_ref):
    sums = jnp.sum(ps_ref[...], axis=0)
    c = jnp.sum(cnt_ref[...], axis=0)[:, 0:1]
    mean = sums / c
    o_ref[0] = w1_ref[0] * jnp.sin(mean) + w2_ref[0] * jnp.cos(mean)


_wave_call = pl.pallas_call(
    _wave_body,
    grid=(_BH,),
    in_specs=[
        pl.BlockSpec(memory_space=pltpu.SMEM),
        pl.BlockSpec(memory_space=pltpu.SMEM),
        pl.BlockSpec((_WPB, _P, _D), lambda b: (b, 0, 0)),
        pl.BlockSpec((_WPB, _P, _LANES), lambda b: (b, 0, 0)),
    ],
    out_specs=pl.BlockSpec((1, _P, _D), lambda b: (b, 0, 0)),
    out_shape=jax.ShapeDtypeStruct((_BH, _P, _D), jnp.float32),
)


def kernel(x, coords, indices, patch_seq_len, conv_w, conv_b, lin1_w, lin1_b,
           lin2_w, lin2_b, wave_w1, wave_w2):
    cw = conv_w[:, 0, :]
    taps = cw + jnp.zeros((2, 5), cw.dtype).at[:, 2].set(1.0)
    wc = taps.T.reshape(10, 1) * jnp.tile(lin1_w, (5, 1))
    b1 = (conv_b @ lin1_w + lin1_b).reshape(1, _D)
    l2w = lin2_w
    l2b = lin2_b.reshape(1, _D)
    rep = jnp.tile(jnp.eye(_DIMS, dtype=jnp.float32), (1, _NUM_HEADS))

    seg = indices[..., 0].astype(jnp.int32)
    seg = seg - (jnp.asarray(patch_seq_len, jnp.int32) - _P)

    c0 = coords[:, :, 0].reshape(_B, 1, _N)
    c1 = coords[:, :, 1].reshape(_B, 1, _N)
    xw = _xw_call(c0, c1, x, wc, b1, l2w, l2b, rep)
    ps, cnt = _seg_partial(xw.reshape(_ROWS_H // 2, 2, _D),
                           seg.reshape(_ROWS_H))
    ps = ps.reshape(_NW, _P, _D)
    cnt = cnt.reshape(_NW, _P, _LANES)
    return _wave_call(wave_w1, wave_w2, ps, cnt)
